# baseline (speedup 1.0000x reference)
import functools

import jax
import jax.numpy as jnp
from jax.experimental import pallas as pl
from jax.experimental.pallas import tpu as pltpu

_NV = 100000
_BLK = 4000


def _moments0_kernel(d_ref, g_ref, s_ref):
    @pl.when(pl.program_id(0) == 0)
    def _init():
        g_ref[...] = jnp.zeros_like(g_ref)
        s_ref[...] = jnp.zeros_like(s_ref)

    d = d_ref[...]
    g_ref[...] += jax.lax.dot_general(
        d, d, (((0,), (0,)), ((), ())), preferred_element_type=jnp.float32)
    s_ref[...] += jnp.sum(d, axis=0, keepdims=True)


def _moments1_kernel(d_ref, a0_ref, c0_ref, m_ref, s_ref):
    @pl.when(pl.program_id(0) == 0)
    def _init():
        m_ref[...] = jnp.zeros_like(m_ref)
        s_ref[...] = jnp.zeros_like(s_ref)

    h = jnp.maximum(
        jnp.dot(d_ref[...], a0_ref[...], preferred_element_type=jnp.float32)
        + c0_ref[...], 0.0)
    m_ref[...] += jax.lax.dot_general(
        h, h, (((0,), (0,)), ((), ())), preferred_element_type=jnp.float32)
    s_ref[...] += jnp.sum(h, axis=0, keepdims=True)


def _mlp_kernel(d_ref, a0_ref, c0_ref, a1_ref, c1_ref, w2_ref, b2_ref, y_ref):
    h = jnp.maximum(
        jnp.dot(d_ref[...], a0_ref[...], preferred_element_type=jnp.float32)
        + c0_ref[...], 0.0)
    h = jnp.maximum(
        jnp.dot(h, a1_ref[...], preferred_element_type=jnp.float32)
        + c1_ref[...], 0.0)
    y_ref[...] = jnp.maximum(
        jnp.dot(h, w2_ref[...], preferred_element_type=jnp.float32)
        + b2_ref[...], 0.0)


def _row_spec(nc):
    return pl.BlockSpec((_BLK, nc), lambda i: (i, 0))


def _full_spec(shape):
    return pl.BlockSpec(shape, lambda i: tuple(0 for _ in shape))


def _fold(W, gn_w, gn_b, mean, var, eps=1e-5):
    inv = gn_w / jnp.sqrt(var + eps)
    return W * inv[None, :], gn_b - mean * inv


def kernel(distributed, indices, W0, gn0_w, gn0_b, W1, gn1_w, gn1_b, W2, b2):
    n = distributed.shape[0]
    grid = (n // _BLK,)
    nf = jnp.float32(n)

    G, s = pl.pallas_call(
        _moments0_kernel,
        grid=grid,
        in_specs=[_row_spec(5)],
        out_specs=[_full_spec((5, 5)), _full_spec((1, 5))],
        out_shape=[
            jax.ShapeDtypeStruct((5, 5), jnp.float32),
            jax.ShapeDtypeStruct((1, 5), jnp.float32),
        ],
    )(distributed)
    mu_x = s[0, :4] / nf
    M0 = G[:4, :4] / nf
    mean0 = mu_x @ W0
    Ey2 = jnp.einsum("aj,ab,bj->j", W0, M0, W0)
    var0 = Ey2 - mean0 * mean0
    A0, c0 = _fold(W0, gn0_w, gn0_b, mean0, var0)
    A0p = jnp.concatenate([A0, jnp.zeros((1, 32), jnp.float32)], axis=0)
    c0r = c0[None, :]

    M1, s1 = pl.pallas_call(
        _moments1_kernel,
        grid=grid,
        in_specs=[_row_spec(5), _full_spec((5, 32)), _full_spec((1, 32))],
        out_specs=[_full_spec((32, 32)), _full_spec((1, 32))],
        out_shape=[
            jax.ShapeDtypeStruct((32, 32), jnp.float32),
            jax.ShapeDtypeStruct((1, 32), jnp.float32),
        ],
    )(distributed, A0p, c0r)
    mu_h = s1[0] / nf
    Mh = M1 / nf
    mean1 = mu_h @ W1
    Ey2_1 = jnp.einsum("aj,ab,bj->j", W1, Mh, W1)
    var1 = Ey2_1 - mean1 * mean1
    A1, c1 = _fold(W1, gn1_w, gn1_b, mean1, var1)
    c1r = c1[None, :]

    y = pl.pallas_call(
        _mlp_kernel,
        grid=grid,
        in_specs=[
            _row_spec(5), _full_spec((5, 32)), _full_spec((1, 32)),
            _full_spec((32, 32)), _full_spec((1, 32)),
            _full_spec((32, 64)), _full_spec((1, 64)),
        ],
        out_specs=_row_spec(64),
        out_shape=jax.ShapeDtypeStruct((n, 64), jnp.float32),
    )(distributed, A0p, c0r, A1, c1r, W2, b2[None, :])

    reduced = jax.ops.segment_max(y, indices, num_segments=_NV)
    reduced = jnp.where(jnp.isneginf(reduced), 0.0, reduced)
    return reduced
